# Initial kernel scaffold; baseline (speedup 1.0000x reference)
import functools

import jax
import jax.numpy as jnp
from jax import lax
from jax.experimental import pallas as pl
from jax.experimental.pallas import tpu as pltpu
from jax.experimental.pallas import tpu_sc as plsc

_VOCAB = 100000
_HIDDEN = 128
_PAD_IDX = 1
_EPS = 1e-12
_B, _S = 1024, 200
_N = _B * _S
_NC, _NS = 2, 16
_NW = _NC * _NS
_TPW = _N // _NW
_GRP = 64
_NGRP = _TPW // _GRP
_NBUF = 4

_TYPE_OFF = 256
_ITEM_OFF = 264
_CTAB = 384


def _sc_gather_body(ids_hbm, table_hbm, out_hbm, idx_v, buf_v,
                    sem0, sem1, sem2, sem3):
    sems = (sem0, sem1, sem2, sem3)
    wid = lax.axis_index("s") * _NC + lax.axis_index("c")
    base_grp = wid * _NGRP
    pltpu.sync_copy(ids_hbm.at[pl.ds(base_grp, _NGRP)], idx_v)

    def _gather(g, b):
        pltpu.make_async_copy(
            table_hbm.at[idx_v.at[g]], buf_v.at[b], sems[b]).start()

    for b in range(_NBUF):
        _gather(b, b)

    def _outer(t, carry):
        for b in range(_NBUF):
            g = t * _NBUF + b
            pltpu.make_async_copy(
                table_hbm.at[idx_v.at[g]], buf_v.at[b], sems[b]).wait()
            pltpu.sync_copy(
                buf_v.at[b],
                out_hbm.at[pl.ds(wid * _TPW + g * _GRP, _GRP)])

            @pl.when(g + _NBUF < _NGRP)
            def _():
                _gather(g + _NBUF, b)
        return carry

    lax.fori_loop(0, _NGRP // _NBUF, _outer, 0)


_sc_gather = pl.kernel(
    _sc_gather_body,
    out_type=jax.ShapeDtypeStruct((_N, _HIDDEN), jnp.float32),
    mesh=plsc.VectorSubcoreMesh(
        core_axis_name="c", subcore_axis_name="s",
        num_cores=_NC, num_subcores=_NS),
    scratch_types=[
        pltpu.VMEM((_NGRP, _GRP), jnp.int32),
        pltpu.VMEM((_NBUF, _GRP, _HIDDEN), jnp.float32),
        pltpu.SemaphoreType.DMA,
        pltpu.SemaphoreType.DMA,
        pltpu.SemaphoreType.DMA,
        pltpu.SemaphoreType.DMA,
    ],
)


_BB = 32


def _tc_body(ids_ref, tt_ref, it_ref, rows_ref, ctab_ref, w_ref, b_ref,
             o_ref):
    ids = ids_ref[...]
    mask = ids != _PAD_IDX
    maskf = mask.astype(jnp.float32)
    ki = lax.broadcasted_iota(jnp.int32, (_S, _S), 0)
    si = lax.broadcasted_iota(jnp.int32, (_S, _S), 1)
    tri = (ki <= si).astype(jnp.float32)
    inc = lax.dot_general(maskf, tri, (((1,), (0,)), ((), ())),
                          preferred_element_type=jnp.float32)
    pid = inc.astype(jnp.int32) * mask.astype(jnp.int32) + _PAD_IDX

    nt = _BB * _S
    col = lax.broadcasted_iota(jnp.int32, (nt, _CTAB), 1)
    pcol = pid.reshape(nt, 1)
    tcol = tt_ref[...].reshape(nt, 1) + _TYPE_OFF
    icol = it_ref[...].reshape(nt, 1) + _ITEM_OFF
    onehot3 = ((col == pcol) | (col == tcol) | (col == icol)).astype(
        jnp.bfloat16)
    extras = lax.dot_general(onehot3, ctab_ref[...], (((1,), (0,)), ((), ())),
                             preferred_element_type=jnp.float32)

    emb = rows_ref[...].reshape(nt, _HIDDEN) + extras
    mean = jnp.mean(emb, axis=1, keepdims=True)
    xc = emb - mean
    var = jnp.mean(xc * xc, axis=1, keepdims=True)
    normed = xc * lax.rsqrt(var + _EPS)
    out = normed * w_ref[...] + b_ref[...]
    o_ref[...] = out.reshape(_BB, _S, _HIDDEN)


def _tc_finish(input_ids, token_type_ids, item_position_ids, word_rows,
               ctab, ln_w, ln_b):
    grid = (_B // _BB,)
    return pl.pallas_call(
        _tc_body,
        grid=grid,
        in_specs=[
            pl.BlockSpec((_BB, _S), lambda i: (i, 0)),
            pl.BlockSpec((_BB, _S), lambda i: (i, 0)),
            pl.BlockSpec((_BB, _S), lambda i: (i, 0)),
            pl.BlockSpec((_BB, _S, _HIDDEN), lambda i: (i, 0, 0)),
            pl.BlockSpec((_CTAB, _HIDDEN), lambda i: (0, 0)),
            pl.BlockSpec((1, _HIDDEN), lambda i: (0, 0)),
            pl.BlockSpec((1, _HIDDEN), lambda i: (0, 0)),
        ],
        out_specs=pl.BlockSpec((_BB, _S, _HIDDEN), lambda i: (i, 0, 0)),
        out_shape=jax.ShapeDtypeStruct((_B, _S, _HIDDEN), jnp.float32),
    )(input_ids, token_type_ids, item_position_ids, word_rows, ctab,
      ln_w, ln_b)


def kernel(input_ids, token_type_ids, item_position_ids, word_embeddings,
           position_embeddings, token_type_embeddings,
           item_position_embeddings, ln_weight, ln_bias):
    ids2d = input_ids.reshape(_N // _GRP, _GRP)
    word_rows = _sc_gather(ids2d, word_embeddings)
    ctab = jnp.concatenate([
        position_embeddings[:_TYPE_OFF],
        token_type_embeddings,
        jnp.zeros((_ITEM_OFF - _TYPE_OFF - 4, _HIDDEN), jnp.float32),
        item_position_embeddings,
        jnp.zeros((_CTAB - _ITEM_OFF - 52, _HIDDEN), jnp.float32),
    ], axis=0).astype(jnp.bfloat16)
    out = _tc_finish(input_ids, token_type_ids, item_position_ids,
                     word_rows.reshape(_B, _S, _HIDDEN), ctab,
                     ln_weight.reshape(1, _HIDDEN),
                     ln_bias.reshape(1, _HIDDEN))
    return out


# speedup vs baseline: 7.0985x; 7.0985x over previous
import functools

import jax
import jax.numpy as jnp
from jax import lax
from jax.experimental import pallas as pl
from jax.experimental.pallas import tpu as pltpu
from jax.experimental.pallas import tpu_sc as plsc

_VOCAB = 100000
_HIDDEN = 128
_PAD_IDX = 1
_EPS = 1e-12
_B, _S = 1024, 200
_N = _B * _S
_NC, _NS = 2, 16
_NW = _NC * _NS
_TPW = _N // _NW
_GRP = 64
_NGRP = _TPW // _GRP
_NBUF = 4

_TYPE_OFF = 256
_ITEM_OFF = 264
_CTAB = 384


def _sc_gather_body(ids_hbm, table_hbm, out_hbm, idx_v, buf_v,
                    sem0, sem1, sem2, sem3):
    sems = (sem0, sem1, sem2, sem3)
    wid = lax.axis_index("s") * _NC + lax.axis_index("c")
    pltpu.sync_copy(ids_hbm.at[wid], idx_v)

    def _gather(g, b):
        pltpu.make_async_copy(
            table_hbm.at[idx_v.at[g]], buf_v.at[b], sems[b]).start()

    for b in range(_NBUF):
        _gather(b, b)

    def _outer(t, carry):
        for b in range(_NBUF):
            g = t * _NBUF + b
            pltpu.make_async_copy(
                table_hbm.at[idx_v.at[g]], buf_v.at[b], sems[b]).wait()
            pltpu.sync_copy(
                buf_v.at[b],
                out_hbm.at[pl.ds(wid * _TPW + g * _GRP, _GRP)])

            @pl.when(g + _NBUF < _NGRP)
            def _():
                _gather(g + _NBUF, b)
        return carry

    lax.fori_loop(0, _NGRP // _NBUF, _outer, 0)


@functools.cache
def _sc_gather():
    return pl.kernel(
        _sc_gather_body,
        out_type=jax.ShapeDtypeStruct((_N, _HIDDEN), jnp.float32),
        mesh=plsc.VectorSubcoreMesh(
            core_axis_name="c", subcore_axis_name="s",
            num_cores=_NC, num_subcores=_NS),
        scratch_types=[
            pltpu.VMEM((_NGRP, _GRP), jnp.int32),
            pltpu.VMEM((_NBUF, _GRP, _HIDDEN), jnp.float32),
            pltpu.SemaphoreType.DMA,
            pltpu.SemaphoreType.DMA,
            pltpu.SemaphoreType.DMA,
            pltpu.SemaphoreType.DMA,
        ],
    )


_PB = 128
_TB = 6400


def _pid_body(ids_ref, pid_ref):
    ids = ids_ref[...]
    mask = ids != _PAD_IDX
    maskf = mask.astype(jnp.float32)
    ki = lax.broadcasted_iota(jnp.int32, (_S, _S), 0)
    si = lax.broadcasted_iota(jnp.int32, (_S, _S), 1)
    tri = (ki <= si).astype(jnp.float32)
    inc = lax.dot_general(maskf, tri, (((1,), (0,)), ((), ())),
                          preferred_element_type=jnp.float32)
    pid_ref[...] = inc.astype(jnp.int32) * mask.astype(jnp.int32) + _PAD_IDX


def _position_ids(input_ids):
    return pl.pallas_call(
        _pid_body,
        grid=(_B // _PB,),
        in_specs=[pl.BlockSpec((_PB, _S), lambda i: (i, 0))],
        out_specs=pl.BlockSpec((_PB, _S), lambda i: (i, 0)),
        out_shape=jax.ShapeDtypeStruct((_B, _S), jnp.int32),
    )(input_ids)


def _tc_body(pid_ref, tt_ref, it_ref, rows_ref, ctab_ref, w_ref, b_ref,
             o_ref):
    col = lax.broadcasted_iota(jnp.int32, (_TB, _CTAB), 1)
    pcol = pid_ref[...]
    tcol = tt_ref[...] + _TYPE_OFF
    icol = it_ref[...] + _ITEM_OFF
    onehot3 = ((col == pcol) | (col == tcol) | (col == icol)).astype(
        jnp.bfloat16)
    extras = lax.dot_general(onehot3, ctab_ref[...], (((1,), (0,)), ((), ())),
                             preferred_element_type=jnp.float32)

    emb = rows_ref[...] + extras
    mean = jnp.mean(emb, axis=1, keepdims=True)
    xc = emb - mean
    var = jnp.mean(xc * xc, axis=1, keepdims=True)
    normed = xc * lax.rsqrt(var + _EPS)
    o_ref[...] = normed * w_ref[...] + b_ref[...]


def _tc_finish(pid_col, tt_col, it_col, word_rows, ctab, ln_w, ln_b):
    return pl.pallas_call(
        _tc_body,
        grid=(_N // _TB,),
        in_specs=[
            pl.BlockSpec((_TB, 1), lambda i: (i, 0)),
            pl.BlockSpec((_TB, 1), lambda i: (i, 0)),
            pl.BlockSpec((_TB, 1), lambda i: (i, 0)),
            pl.BlockSpec((_TB, _HIDDEN), lambda i: (i, 0)),
            pl.BlockSpec((_CTAB, _HIDDEN), lambda i: (0, 0)),
            pl.BlockSpec((1, _HIDDEN), lambda i: (0, 0)),
            pl.BlockSpec((1, _HIDDEN), lambda i: (0, 0)),
        ],
        out_specs=pl.BlockSpec((_TB, _HIDDEN), lambda i: (i, 0)),
        out_shape=jax.ShapeDtypeStruct((_N, _HIDDEN), jnp.float32),
    )(pid_col, tt_col, it_col, word_rows, ctab, ln_w, ln_b)


def kernel(input_ids, token_type_ids, item_position_ids, word_embeddings,
           position_embeddings, token_type_embeddings,
           item_position_embeddings, ln_weight, ln_bias):
    ids3d = input_ids.reshape(_NW, _NGRP, _GRP)
    word_rows = _sc_gather()(ids3d, word_embeddings)
    pid = _position_ids(input_ids)
    ctab = jnp.concatenate([
        position_embeddings[:_TYPE_OFF],
        token_type_embeddings,
        jnp.zeros((_ITEM_OFF - _TYPE_OFF - 4, _HIDDEN), jnp.float32),
        item_position_embeddings,
        jnp.zeros((_CTAB - _ITEM_OFF - 52, _HIDDEN), jnp.float32),
    ], axis=0).astype(jnp.bfloat16)
    out = _tc_finish(pid.reshape(_N, 1), token_type_ids.reshape(_N, 1),
                     item_position_ids.reshape(_N, 1), word_rows, ctab,
                     ln_weight.reshape(1, _HIDDEN),
                     ln_bias.reshape(1, _HIDDEN))
    return out.reshape(_B, _S, _HIDDEN)


# speedup vs baseline: 11.0135x; 1.5515x over previous
import functools

import jax
import jax.numpy as jnp
from jax import lax
from jax.experimental import pallas as pl
from jax.experimental.pallas import tpu as pltpu
from jax.experimental.pallas import tpu_sc as plsc

_VOCAB = 100000
_HIDDEN = 128
_PAD_IDX = 1
_EPS = 1e-12
_B, _S = 1024, 200
_N = _B * _S
_NC, _NS = 2, 16
_NW = _NC * _NS
_TPW = _N // _NW
_GRP = 64
_NGRP = _TPW // _GRP
_NBUF = 4

_CPOS = 256
_CTI = 256


def _sc_gather_body(ids_hbm, table_hbm, out_hbm, idx_v, buf_v,
                    sem0, sem1, sem2, sem3):
    sems = (sem0, sem1, sem2, sem3)
    wid = lax.axis_index("s") * _NC + lax.axis_index("c")
    pltpu.sync_copy(ids_hbm.at[wid], idx_v)

    def _gather(g, b):
        pltpu.make_async_copy(
            table_hbm.at[idx_v.at[g]], buf_v.at[b], sems[b]).start()

    for b in range(_NBUF):
        _gather(b, b)

    def _outer(t, carry):
        for b in range(_NBUF):
            g = t * _NBUF + b
            pltpu.make_async_copy(
                table_hbm.at[idx_v.at[g]], buf_v.at[b], sems[b]).wait()
            pltpu.sync_copy(
                buf_v.at[b],
                out_hbm.at[pl.ds(wid * _TPW + g * _GRP, _GRP)])

            @pl.when(g + _NBUF < _NGRP)
            def _():
                _gather(g + _NBUF, b)
        return carry

    lax.fori_loop(0, _NGRP // _NBUF, _outer, 0)


@functools.cache
def _sc_gather():
    return pl.kernel(
        _sc_gather_body,
        out_type=jax.ShapeDtypeStruct((_N, _HIDDEN), jnp.float32),
        mesh=plsc.VectorSubcoreMesh(
            core_axis_name="c", subcore_axis_name="s",
            num_cores=_NC, num_subcores=_NS),
        scratch_types=[
            pltpu.VMEM((_NGRP, _GRP), jnp.int32),
            pltpu.VMEM((_NBUF, _GRP, _HIDDEN), jnp.float32),
            pltpu.SemaphoreType.DMA,
            pltpu.SemaphoreType.DMA,
            pltpu.SemaphoreType.DMA,
            pltpu.SemaphoreType.DMA,
        ],
    )


_PB = 128
_TB = 6400


def _pid_body(ids_ref, tt_ref, it_ref, packed_ref):
    ids = ids_ref[...]
    mask = ids != _PAD_IDX
    maskf = mask.astype(jnp.float32)
    ki = lax.broadcasted_iota(jnp.int32, (_S, _S), 0)
    si = lax.broadcasted_iota(jnp.int32, (_S, _S), 1)
    tri = (ki <= si).astype(jnp.float32)
    inc = lax.dot_general(maskf, tri, (((1,), (0,)), ((), ())),
                          preferred_element_type=jnp.float32)
    pid = inc.astype(jnp.int32) * mask.astype(jnp.int32) + _PAD_IDX
    ti = tt_ref[...] * 64 + it_ref[...]
    packed_ref[...] = pid + ti * 256


def _position_ids(input_ids, token_type_ids, item_position_ids):
    return pl.pallas_call(
        _pid_body,
        grid=(_B // _PB,),
        in_specs=[pl.BlockSpec((_PB, _S), lambda i: (i, 0))] * 3,
        out_specs=pl.BlockSpec((_PB, _S), lambda i: (i, 0)),
        out_shape=jax.ShapeDtypeStruct((_B, _S), jnp.int32),
    )(input_ids, token_type_ids, item_position_ids)


def _tc_body(idx_ref, rows_ref, ptab_ref, titab_ref, ones_ref, w_ref, b_ref,
             o_ref):
    dn = (((1,), (0,)), ((), ()))
    idx = jnp.transpose(idx_ref[0], (1, 0))
    pcol = jnp.bitwise_and(idx, 255)
    ticol = lax.shift_right_logical(idx, 8)
    col = lax.broadcasted_iota(jnp.int32, (_TB, _CPOS), 1)
    oh_p = (col == pcol).astype(jnp.bfloat16)
    oh_ti = (col == ticol).astype(jnp.bfloat16)
    extras = (
        lax.dot_general(oh_p, ptab_ref[...], dn,
                        preferred_element_type=jnp.float32)
        + lax.dot_general(oh_ti, titab_ref[...], dn,
                          preferred_element_type=jnp.float32))

    emb = rows_ref[...] + extras
    ones = ones_ref[...]
    mean = lax.dot_general(emb, ones, dn,
                           preferred_element_type=jnp.float32)[:, 0:1]
    mean = mean * (1.0 / _HIDDEN)
    sumsq = lax.dot_general(emb * emb, ones, dn,
                            preferred_element_type=jnp.float32)[:, 0:1]
    var = sumsq * (1.0 / _HIDDEN) - mean * mean
    normed = (emb - mean) * lax.rsqrt(var + _EPS)
    o_ref[...] = normed * w_ref[...] + b_ref[...]


def _tc_finish(idsx, word_rows, ptab, titab, ones, ln_w, ln_b):
    return pl.pallas_call(
        _tc_body,
        grid=(_N // _TB,),
        in_specs=[
            pl.BlockSpec((1, 1, _TB), lambda i: (i, 0, 0)),
            pl.BlockSpec((_TB, _HIDDEN), lambda i: (i, 0)),
            pl.BlockSpec((_CPOS, _HIDDEN), lambda i: (0, 0)),
            pl.BlockSpec((_CTI, _HIDDEN), lambda i: (0, 0)),
            pl.BlockSpec((_HIDDEN, 8), lambda i: (0, 0)),
            pl.BlockSpec((1, _HIDDEN), lambda i: (0, 0)),
            pl.BlockSpec((1, _HIDDEN), lambda i: (0, 0)),
        ],
        out_specs=pl.BlockSpec((_TB, _HIDDEN), lambda i: (i, 0)),
        out_shape=jax.ShapeDtypeStruct((_N, _HIDDEN), jnp.float32),
    )(idsx, word_rows, ptab, titab, ones, ln_w, ln_b)


def kernel(input_ids, token_type_ids, item_position_ids, word_embeddings,
           position_embeddings, token_type_embeddings,
           item_position_embeddings, ln_weight, ln_bias):
    ids3d = input_ids.reshape(_NW, _NGRP, _GRP)
    word_rows = _sc_gather()(ids3d, word_embeddings)
    packed = _position_ids(input_ids, token_type_ids, item_position_ids)
    ptab = position_embeddings[:_CPOS].astype(jnp.bfloat16)
    ipad = jnp.concatenate([
        item_position_embeddings,
        jnp.zeros((64 - 52, _HIDDEN), jnp.float32),
    ], axis=0)
    titab = (token_type_embeddings[:, None, :]
             + ipad[None, :, :]).reshape(_CTI, _HIDDEN).astype(jnp.bfloat16)
    idsx = packed.reshape(_N // _TB, 1, _TB)
    ones = jnp.ones((_HIDDEN, 8), jnp.float32)
    out = _tc_finish(idsx, word_rows, ptab, titab, ones,
                     ln_weight.reshape(1, _HIDDEN),
                     ln_bias.reshape(1, _HIDDEN))
    return out.reshape(_B, _S, _HIDDEN)


# speedup vs baseline: 11.6664x; 1.0593x over previous
import functools

import jax
import jax.numpy as jnp
from jax import lax
from jax.experimental import pallas as pl
from jax.experimental.pallas import tpu as pltpu
from jax.experimental.pallas import tpu_sc as plsc

_VOCAB = 100000
_HIDDEN = 128
_PAD_IDX = 1
_EPS = 1e-12
_B, _S = 1024, 200
_N = _B * _S
_K = 4
_NCH = _N // _K
_NC, _NS = 2, 16
_NW = _NC * _NS
_TPW = _NCH // _NW
_GRP = 64
_NGRP = _TPW // _GRP
_NBUF = 5

_CPOS = 256
_CTI = 256

_PB = 128
_TB = 6400


def _sc_gather_body(ids_hbm, table_hbm, out_hbm, idx_v, buf_v,
                    sem0, sem1, sem2, sem3, sem4):
    sems = (sem0, sem1, sem2, sem3, sem4)
    wid = lax.axis_index("s") * _NC + lax.axis_index("c")
    pltpu.sync_copy(ids_hbm.at[wid], idx_v)

    def _gather(g, b):
        pltpu.make_async_copy(
            table_hbm.at[idx_v.at[g]], buf_v.at[b], sems[b]).start()

    for b in range(_NBUF):
        _gather(b, b)

    def _outer(t, carry):
        for b in range(_NBUF):
            g = t * _NBUF + b
            pltpu.make_async_copy(
                table_hbm.at[idx_v.at[g]], buf_v.at[b], sems[b]).wait()
            pltpu.sync_copy(
                buf_v.at[b],
                out_hbm.at[pl.ds(wid * _TPW + g * _GRP, _GRP)])

            @pl.when(g + _NBUF < _NGRP)
            def _():
                _gather(g + _NBUF, b)
        return carry

    lax.fori_loop(0, _NGRP // _NBUF, _outer, 0)


@functools.cache
def _sc_gather():
    return pl.kernel(
        _sc_gather_body,
        out_type=jax.ShapeDtypeStruct((_NCH, _HIDDEN), jnp.float32),
        mesh=plsc.VectorSubcoreMesh(
            core_axis_name="c", subcore_axis_name="s",
            num_cores=_NC, num_subcores=_NS),
        scratch_types=[
            pltpu.VMEM((_NGRP, _GRP), jnp.int32),
            pltpu.VMEM((_NBUF, _GRP, _HIDDEN), jnp.float32),
            pltpu.SemaphoreType.DMA,
            pltpu.SemaphoreType.DMA,
            pltpu.SemaphoreType.DMA,
            pltpu.SemaphoreType.DMA,
            pltpu.SemaphoreType.DMA,
        ],
    )


def _pid_body(ids_ref, tt_ref, it_ref, packed_ref):
    ids = ids_ref[...]
    mask = ids != _PAD_IDX
    maskf = mask.astype(jnp.float32)
    ki = lax.broadcasted_iota(jnp.int32, (_S, _S), 0)
    si = lax.broadcasted_iota(jnp.int32, (_S, _S), 1)
    tri = (ki <= si).astype(jnp.float32)
    inc = lax.dot_general(maskf, tri, (((1,), (0,)), ((), ())),
                          preferred_element_type=jnp.float32)
    pid = inc.astype(jnp.int32) * mask.astype(jnp.int32) + _PAD_IDX
    ti = tt_ref[...] * 64 + it_ref[...]
    packed_ref[...] = pid + ti * 256


def _position_ids(input_ids, token_type_ids, item_position_ids):
    return pl.pallas_call(
        _pid_body,
        grid=(_B // _PB,),
        in_specs=[pl.BlockSpec((_PB, _S), lambda i: (i, 0))] * 3,
        out_specs=pl.BlockSpec((_PB, _S), lambda i: (i, 0)),
        out_shape=jax.ShapeDtypeStruct((_B, _S), jnp.int32),
    )(input_ids, token_type_ids, item_position_ids)


def _tc_body(idx_ref, rows_ref, ptab_ref, titab_ref, ones_ref, w_ref, b_ref,
             o_ref):
    dn = (((1,), (0,)), ((), ()))
    idx = jnp.transpose(idx_ref[0], (1, 0))
    pcol = jnp.bitwise_and(idx, 255)
    ticol = lax.shift_right_logical(idx, 8)
    col = lax.broadcasted_iota(jnp.int32, (_TB, _CPOS), 1)
    oh_p = (col == pcol).astype(jnp.bfloat16)
    oh_ti = (col == ticol).astype(jnp.bfloat16)
    extras = (
        lax.dot_general(oh_p, ptab_ref[...], dn,
                        preferred_element_type=jnp.float32)
        + lax.dot_general(oh_ti, titab_ref[...], dn,
                          preferred_element_type=jnp.float32))

    emb = rows_ref[...] + extras
    ones = ones_ref[...]
    mean = lax.dot_general(emb, ones, dn,
                           preferred_element_type=jnp.float32)[:, 0:1]
    mean = mean * (1.0 / _HIDDEN)
    sumsq = lax.dot_general(emb * emb, ones, dn,
                            preferred_element_type=jnp.float32)[:, 0:1]
    var = sumsq * (1.0 / _HIDDEN) - mean * mean
    normed = (emb - mean) * lax.rsqrt(var + _EPS)
    o_ref[...] = normed * w_ref[...] + b_ref[...]


def _tc_body_acc(acc_ref, idx_ref, rows_ref, ptab_ref, titab_ref, ones_ref,
                 w_ref, b_ref, o_ref):
    del acc_ref
    _tc_body(idx_ref, rows_ref, ptab_ref, titab_ref, ones_ref, w_ref, b_ref,
             o_ref)


def _tc_finish_chunk(k, out_buf, idsx, rows_k, ptab, titab, ones, ln_w, ln_b):
    blk0 = k * (_NCH // _TB)
    common_specs = [
        pl.BlockSpec((1, 1, _TB), lambda i: (blk0 + i, 0, 0)),
        pl.BlockSpec((_TB, _HIDDEN), lambda i: (i, 0)),
        pl.BlockSpec((_CPOS, _HIDDEN), lambda i: (0, 0)),
        pl.BlockSpec((_CTI, _HIDDEN), lambda i: (0, 0)),
        pl.BlockSpec((_HIDDEN, 8), lambda i: (0, 0)),
        pl.BlockSpec((1, _HIDDEN), lambda i: (0, 0)),
        pl.BlockSpec((1, _HIDDEN), lambda i: (0, 0)),
    ]
    out_spec = pl.BlockSpec((_TB, _HIDDEN), lambda i: (blk0 + i, 0))
    out_shape = jax.ShapeDtypeStruct((_N, _HIDDEN), jnp.float32)
    if out_buf is None:
        return pl.pallas_call(
            _tc_body,
            grid=(_NCH // _TB,),
            in_specs=common_specs,
            out_specs=out_spec,
            out_shape=out_shape,
        )(idsx, rows_k, ptab, titab, ones, ln_w, ln_b)
    return pl.pallas_call(
        _tc_body_acc,
        grid=(_NCH // _TB,),
        in_specs=[pl.BlockSpec(memory_space=pl.ANY)] + common_specs,
        out_specs=out_spec,
        out_shape=out_shape,
        input_output_aliases={0: 0},
    )(out_buf, idsx, rows_k, ptab, titab, ones, ln_w, ln_b)


def kernel(input_ids, token_type_ids, item_position_ids, word_embeddings,
           position_embeddings, token_type_embeddings,
           item_position_embeddings, ln_weight, ln_bias):
    ids4d = input_ids.reshape(_K, _NW, _NGRP, _GRP)
    gather = _sc_gather()
    rows = [gather(ids4d[k], word_embeddings) for k in range(_K)]

    packed = _position_ids(input_ids, token_type_ids, item_position_ids)
    idsx = packed.reshape(_N // _TB, 1, _TB)
    ptab = position_embeddings[:_CPOS].astype(jnp.bfloat16)
    ipad = jnp.concatenate([
        item_position_embeddings,
        jnp.zeros((64 - 52, _HIDDEN), jnp.float32),
    ], axis=0)
    titab = (token_type_embeddings[:, None, :]
             + ipad[None, :, :]).reshape(_CTI, _HIDDEN).astype(jnp.bfloat16)
    ones = jnp.ones((_HIDDEN, 8), jnp.float32)
    ln_w = ln_weight.reshape(1, _HIDDEN)
    ln_b = ln_bias.reshape(1, _HIDDEN)

    out = None
    for k in range(_K):
        out = _tc_finish_chunk(k, out, idsx, rows[k], ptab, titab, ones,
                               ln_w, ln_b)
    return out.reshape(_B, _S, _HIDDEN)
